# baseline (speedup 1.0000x reference)
import functools

import jax
import jax.numpy as jnp
from jax import lax
from jax.experimental import pallas as pl
from jax.experimental.pallas import tpu as pltpu
from jax.experimental.pallas import tpu_sc as plsc

VOCAB = 100000
EMBED = 256
HIDDEN = 1024
FC1 = 128
OUT = 2
B = 1024
L = 20
LANE = 128
SPLIT = 6


@functools.cache
def _make_sc_gather(V, D, N):
    info = plsc.get_sparse_core_info()
    nw = info.num_cores * info.num_subcores
    n_per_w = N // nw
    assert N % (8 * nw) == 0
    ch = 8
    for c in range(8, 129, 8):
        if n_per_w % c == 0:
            ch = c
    n_ch = n_per_w // ch
    mesh = plsc.VectorSubcoreMesh(core_axis_name="c", subcore_axis_name="s")

    @functools.partial(
        pl.kernel,
        mesh=mesh,
        out_type=jax.ShapeDtypeStruct((N, D), jnp.float32),
        scratch_types=[
            pltpu.VMEM((2, ch), jnp.int32),
            pltpu.VMEM((2, ch, D), jnp.float32),
            pltpu.SemaphoreType.DMA,
            pltpu.SemaphoreType.DMA,
            pltpu.SemaphoreType.DMA,
        ],
    )
    def gather(table_hbm, idx_hbm, out_hbm, idx_v, rows_v, isem, gsem, osem):
        wid = lax.axis_index("s") * info.num_cores + lax.axis_index("c")
        base = wid * n_per_w
        out_cp = [None] * n_ch
        pltpu.sync_copy(idx_hbm.at[pl.ds(base, ch)], idx_v.at[0])
        for c in range(n_ch):
            s = c % 2
            if c + 1 < n_ch:
                icp = pltpu.async_copy(
                    idx_hbm.at[pl.ds(base + (c + 1) * ch, ch)],
                    idx_v.at[1 - s], isem)
            if c >= 2:
                out_cp[c - 2].wait()
            pltpu.async_copy(table_hbm.at[idx_v.at[s]], rows_v.at[s],
                             gsem).wait()
            out_cp[c] = pltpu.async_copy(
                rows_v.at[s], out_hbm.at[pl.ds(base + c * ch, ch)], osem)
            if c + 1 < n_ch:
                icp.wait()
        for c in range(max(n_ch - 2, 0), n_ch):
            out_cp[c].wait()

    return gather


_NT = (((1,), (1,)), ((), ()))


SPI1 = 3
SPI2 = 7


def _step(emb_blk, h_prev, wih_ref, whh_ref, bias_ref):
    acc = lax.dot_general(emb_blk.astype(jnp.bfloat16), wih_ref[...],
                          _NT, preferred_element_type=jnp.float32)
    acc = acc + lax.dot_general(h_prev, whh_ref[...], _NT,
                                preferred_element_type=jnp.float32)
    return jnp.tanh(acc + bias_ref[...])


def _rnn_seg1_body(emb_ref, wih_ref, whh_ref, bias_ref, out_ref, h_ref):
    t = pl.program_id(0)

    @pl.when(t == 0)
    def _():
        h_ref[...] = jnp.zeros_like(h_ref)

    h = h_ref[...]
    for s in range(SPI1):
        h = _step(emb_ref[pl.ds(s * B, B), :], h, wih_ref, whh_ref,
                  bias_ref).astype(jnp.bfloat16)
    h_ref[...] = h

    @pl.when(t == SPLIT // SPI1 - 1)
    def _():
        out_ref[...] = h


def _rnn_seg2_body(emb_ref, h0_ref, wih_ref, whh_ref, bias_ref, fc1w_ref,
                   fc1b_ref, fc2w_ref, fc2b_ref, out_ref, h_ref):
    t = pl.program_id(0)
    T = (L - SPLIT) // SPI2

    @pl.when(t == 0)
    def _():
        h_ref[...] = h0_ref[...]

    h = h_ref[...]
    for s in range(SPI2):
        h_new = _step(emb_ref[pl.ds(s * B, B), :], h, wih_ref, whh_ref,
                      bias_ref)
        h = h_new.astype(jnp.bfloat16)
    h_ref[...] = h

    @pl.when(t == T - 1)
    def _():
        feat = lax.dot_general(h_new, fc1w_ref[...], _NT,
                               preferred_element_type=jnp.float32)
        feat = jnp.maximum(feat + fc1b_ref[...], 0.0)
        logits = lax.dot_general(feat, fc2w_ref[...], _NT,
                                 preferred_element_type=jnp.float32)
        logits = logits + fc2b_ref[...]
        m = jnp.max(logits, axis=1, keepdims=True)
        e = jnp.exp(logits - m)
        p = e / jnp.sum(e, axis=1, keepdims=True)
        out_ref[...] = p[:, :OUT]


_FULL = lambda t: (0, 0)


def _rnn_seg1(emb1, wih, whh, bias):
    return pl.pallas_call(
        _rnn_seg1_body,
        grid=(SPLIT // SPI1,),
        in_specs=[
            pl.BlockSpec((SPI1 * B, EMBED), lambda t: (t, 0)),
            pl.BlockSpec((HIDDEN, EMBED), _FULL),
            pl.BlockSpec((HIDDEN, HIDDEN), _FULL),
            pl.BlockSpec((1, HIDDEN), _FULL),
        ],
        out_specs=pl.BlockSpec((B, HIDDEN), _FULL),
        out_shape=jax.ShapeDtypeStruct((B, HIDDEN), jnp.bfloat16),
        scratch_shapes=[pltpu.VMEM((B, HIDDEN), jnp.bfloat16)],
        compiler_params=pltpu.CompilerParams(
            dimension_semantics=("arbitrary",)),
    )(emb1, wih, whh, bias)


def _rnn_seg2(emb2, h0, wih, whh, bias, fc1w_t, fc1b, fc2w_pad, fc2b_pad):
    return pl.pallas_call(
        _rnn_seg2_body,
        grid=((L - SPLIT) // SPI2,),
        in_specs=[
            pl.BlockSpec((SPI2 * B, EMBED), lambda t: (t, 0)),
            pl.BlockSpec((B, HIDDEN), _FULL),
            pl.BlockSpec((HIDDEN, EMBED), _FULL),
            pl.BlockSpec((HIDDEN, HIDDEN), _FULL),
            pl.BlockSpec((1, HIDDEN), _FULL),
            pl.BlockSpec((FC1, HIDDEN), _FULL),
            pl.BlockSpec((1, FC1), _FULL),
            pl.BlockSpec((LANE, FC1), _FULL),
            pl.BlockSpec((1, LANE), _FULL),
        ],
        out_specs=pl.BlockSpec((B, OUT), _FULL),
        out_shape=jax.ShapeDtypeStruct((B, OUT), jnp.float32),
        scratch_shapes=[pltpu.VMEM((B, HIDDEN), jnp.bfloat16)],
        compiler_params=pltpu.CompilerParams(
            dimension_semantics=("arbitrary",)),
    )(emb2, h0, wih, whh, bias, fc1w_t, fc1b, fc2w_pad, fc2b_pad)


def kernel(x, embed_table, W_ih, b_ih, W_hh, b_hh, fc1_W, fc1_b, fc2_W, fc2_b):
    idx = jnp.swapaxes(x, 0, 1).reshape(-1).astype(jnp.int32)
    idx1 = idx[: SPLIT * B]
    idx2 = idx[SPLIT * B:]
    emb1 = _make_sc_gather(VOCAB, EMBED, SPLIT * B)(embed_table, idx1)
    emb2 = _make_sc_gather(VOCAB, EMBED, (L - SPLIT) * B)(embed_table, idx2)

    wih = W_ih.astype(jnp.bfloat16)
    whh = W_hh.astype(jnp.bfloat16)
    bias = (b_ih + b_hh).reshape(1, HIDDEN)
    fc2w_pad = jnp.pad(fc2_W, ((0, LANE - OUT), (0, 0)))
    fc2b_pad = jnp.pad(fc2_b, (0, LANE - OUT),
                       constant_values=-1e30).reshape(1, LANE)

    h_mid = _rnn_seg1(emb1, wih, whh, bias)
    return _rnn_seg2(emb2, h_mid, wih, whh, bias, fc1_W,
                     fc1_b.reshape(1, FC1), fc2w_pad, fc2b_pad)


# speedup vs baseline: 1.0018x; 1.0018x over previous
import functools

import jax
import jax.numpy as jnp
from jax import lax
from jax.experimental import pallas as pl
from jax.experimental.pallas import tpu as pltpu
from jax.experimental.pallas import tpu_sc as plsc

VOCAB = 100000
EMBED = 256
HIDDEN = 1024
FC1 = 128
OUT = 2
B = 1024
L = 20
LANE = 128
SPLIT = 6


@functools.cache
def _make_sc_gather(V, D, N):
    info = plsc.get_sparse_core_info()
    nw = info.num_cores * info.num_subcores
    n_per_w = N // nw
    assert N % (8 * nw) == 0
    ch = 8
    for c in range(8, 129, 8):
        if n_per_w % c == 0:
            ch = c
    n_ch = n_per_w // ch
    mesh = plsc.VectorSubcoreMesh(core_axis_name="c", subcore_axis_name="s")

    @functools.partial(
        pl.kernel,
        mesh=mesh,
        out_type=jax.ShapeDtypeStruct((N, D), jnp.float32),
        scratch_types=[
            pltpu.VMEM((2, ch), jnp.int32),
            pltpu.VMEM((2, ch, D), jnp.float32),
            pltpu.SemaphoreType.DMA,
            pltpu.SemaphoreType.DMA,
            pltpu.SemaphoreType.DMA,
        ],
    )
    def gather(table_hbm, idx_hbm, out_hbm, idx_v, rows_v, isem, gsem, osem):
        wid = lax.axis_index("s") * info.num_cores + lax.axis_index("c")
        base = wid * n_per_w
        out_cp = [None] * n_ch
        pltpu.sync_copy(idx_hbm.at[pl.ds(base, ch)], idx_v.at[0])
        for c in range(n_ch):
            s = c % 2
            if c + 1 < n_ch:
                icp = pltpu.async_copy(
                    idx_hbm.at[pl.ds(base + (c + 1) * ch, ch)],
                    idx_v.at[1 - s], isem)
            if c >= 2:
                out_cp[c - 2].wait()
            pltpu.async_copy(table_hbm.at[idx_v.at[s]], rows_v.at[s],
                             gsem).wait()
            out_cp[c] = pltpu.async_copy(
                rows_v.at[s], out_hbm.at[pl.ds(base + c * ch, ch)], osem)
            if c + 1 < n_ch:
                icp.wait()
        for c in range(max(n_ch - 2, 0), n_ch):
            out_cp[c].wait()

    return gather


_NT = (((1,), (1,)), ((), ()))


SPI1 = 3
SPI2 = 7


def _step(emb_blk, h_prev, wih_ref, whh_ref, bias_ref):
    acc = lax.dot_general(emb_blk.astype(jnp.bfloat16), wih_ref[...],
                          _NT, preferred_element_type=jnp.float32)
    acc = acc + lax.dot_general(h_prev, whh_ref[...], _NT,
                                preferred_element_type=jnp.float32)
    return jnp.tanh(acc + bias_ref[...])


def _rnn_seg1_body(emb_ref, wih_ref, whh_ref, bias_ref, out_ref, h_ref):
    t = pl.program_id(0)
    h = jnp.where(t > 0, h_ref[...], jnp.zeros((B, HIDDEN), jnp.bfloat16))
    for s in range(SPI1):
        h = _step(emb_ref[pl.ds(s * B, B), :], h, wih_ref, whh_ref,
                  bias_ref).astype(jnp.bfloat16)
    h_ref[...] = h

    @pl.when(t == SPLIT // SPI1 - 1)
    def _():
        out_ref[...] = h


def _rnn_seg2_body(emb_ref, h0_ref, wih_ref, whh_ref, bias_ref, fc1w_ref,
                   fc1b_ref, fc2w_ref, fc2b_ref, out_ref, h_ref):
    t = pl.program_id(0)
    T = (L - SPLIT) // SPI2
    h = jnp.where(t > 0, h_ref[...], h0_ref[...])
    for s in range(SPI2):
        h_new = _step(emb_ref[pl.ds(s * B, B), :], h, wih_ref, whh_ref,
                      bias_ref)
        h = h_new.astype(jnp.bfloat16)
    h_ref[...] = h

    @pl.when(t == T - 1)
    def _():
        feat = lax.dot_general(h_new, fc1w_ref[...], _NT,
                               preferred_element_type=jnp.float32)
        feat = jnp.maximum(feat + fc1b_ref[...], 0.0)
        logits = lax.dot_general(feat, fc2w_ref[...], _NT,
                                 preferred_element_type=jnp.float32)
        logits = logits + fc2b_ref[...]
        m = jnp.max(logits, axis=1, keepdims=True)
        e = jnp.exp(logits - m)
        p = e / jnp.sum(e, axis=1, keepdims=True)
        out_ref[...] = p[:, :OUT]


_FULL = lambda t: (0, 0)


def _rnn_seg1(emb1, wih, whh, bias):
    return pl.pallas_call(
        _rnn_seg1_body,
        grid=(SPLIT // SPI1,),
        in_specs=[
            pl.BlockSpec((SPI1 * B, EMBED), lambda t: (t, 0)),
            pl.BlockSpec((HIDDEN, EMBED), _FULL),
            pl.BlockSpec((HIDDEN, HIDDEN), _FULL),
            pl.BlockSpec((1, HIDDEN), _FULL),
        ],
        out_specs=pl.BlockSpec((B, HIDDEN), _FULL),
        out_shape=jax.ShapeDtypeStruct((B, HIDDEN), jnp.bfloat16),
        scratch_shapes=[pltpu.VMEM((B, HIDDEN), jnp.bfloat16)],
        compiler_params=pltpu.CompilerParams(
            dimension_semantics=("arbitrary",)),
    )(emb1, wih, whh, bias)


def _rnn_seg2(emb2, h0, wih, whh, bias, fc1w_t, fc1b, fc2w_pad, fc2b_pad):
    return pl.pallas_call(
        _rnn_seg2_body,
        grid=((L - SPLIT) // SPI2,),
        in_specs=[
            pl.BlockSpec((SPI2 * B, EMBED), lambda t: (t, 0)),
            pl.BlockSpec((B, HIDDEN), _FULL),
            pl.BlockSpec((HIDDEN, EMBED), _FULL),
            pl.BlockSpec((HIDDEN, HIDDEN), _FULL),
            pl.BlockSpec((1, HIDDEN), _FULL),
            pl.BlockSpec((FC1, HIDDEN), _FULL),
            pl.BlockSpec((1, FC1), _FULL),
            pl.BlockSpec((LANE, FC1), _FULL),
            pl.BlockSpec((1, LANE), _FULL),
        ],
        out_specs=pl.BlockSpec((B, OUT), _FULL),
        out_shape=jax.ShapeDtypeStruct((B, OUT), jnp.float32),
        scratch_shapes=[pltpu.VMEM((B, HIDDEN), jnp.bfloat16)],
        compiler_params=pltpu.CompilerParams(
            dimension_semantics=("arbitrary",)),
    )(emb2, h0, wih, whh, bias, fc1w_t, fc1b, fc2w_pad, fc2b_pad)


def kernel(x, embed_table, W_ih, b_ih, W_hh, b_hh, fc1_W, fc1_b, fc2_W, fc2_b):
    idx = jnp.swapaxes(x, 0, 1).reshape(-1).astype(jnp.int32)
    idx1 = idx[: SPLIT * B]
    idx2 = idx[SPLIT * B:]
    emb1 = _make_sc_gather(VOCAB, EMBED, SPLIT * B)(embed_table, idx1)
    emb2 = _make_sc_gather(VOCAB, EMBED, (L - SPLIT) * B)(embed_table, idx2)

    wih = W_ih.astype(jnp.bfloat16)
    whh = W_hh.astype(jnp.bfloat16)
    bias = (b_ih + b_hh).reshape(1, HIDDEN)
    fc2w_pad = jnp.pad(fc2_W, ((0, LANE - OUT), (0, 0)))
    fc2b_pad = jnp.pad(fc2_b, (0, LANE - OUT),
                       constant_values=-1e30).reshape(1, LANE)

    h_mid = _rnn_seg1(emb1, wih, whh, bias)
    return _rnn_seg2(emb2, h_mid, wih, whh, bias, fc1_W,
                     fc1_b.reshape(1, FC1), fc2w_pad, fc2b_pad)


# speedup vs baseline: 1.0045x; 1.0027x over previous
import functools

import jax
import jax.numpy as jnp
from jax import lax
from jax.experimental import pallas as pl
from jax.experimental.pallas import tpu as pltpu
from jax.experimental.pallas import tpu_sc as plsc

VOCAB = 100000
EMBED = 256
HIDDEN = 1024
FC1 = 128
OUT = 2
B = 1024
L = 20
LANE = 128
SPLIT = 6


@functools.cache
def _make_sc_gather(V, D, N):
    info = plsc.get_sparse_core_info()
    nw = info.num_cores * info.num_subcores
    n_per_w = N // nw
    assert N % (8 * nw) == 0
    ch = 8
    for c in range(8, 129, 8):
        if n_per_w % c == 0:
            ch = c
    n_ch = n_per_w // ch
    mesh = plsc.VectorSubcoreMesh(core_axis_name="c", subcore_axis_name="s")

    @functools.partial(
        pl.kernel,
        mesh=mesh,
        out_type=jax.ShapeDtypeStruct((N, D), jnp.float32),
        scratch_types=[
            pltpu.VMEM((2, ch), jnp.int32),
            pltpu.VMEM((2, ch, D), jnp.float32),
            pltpu.SemaphoreType.DMA,
            pltpu.SemaphoreType.DMA,
            pltpu.SemaphoreType.DMA,
        ],
    )
    def gather(table_hbm, idx_hbm, out_hbm, idx_v, rows_v, isem, gsem, osem):
        wid = lax.axis_index("s") * info.num_cores + lax.axis_index("c")
        base = wid * n_per_w
        out_cp = [None] * n_ch
        pltpu.sync_copy(idx_hbm.at[pl.ds(base, ch)], idx_v.at[0])
        for c in range(n_ch):
            s = c % 2
            if c + 1 < n_ch:
                icp = pltpu.async_copy(
                    idx_hbm.at[pl.ds(base + (c + 1) * ch, ch)],
                    idx_v.at[1 - s], isem)
            if c >= 2:
                out_cp[c - 2].wait()
            pltpu.async_copy(table_hbm.at[idx_v.at[s]], rows_v.at[s],
                             gsem).wait()
            out_cp[c] = pltpu.async_copy(
                rows_v.at[s], out_hbm.at[pl.ds(base + c * ch, ch)], osem)
            if c + 1 < n_ch:
                icp.wait()
        for c in range(max(n_ch - 2, 0), n_ch):
            out_cp[c].wait()

    return gather


_NT = (((1,), (1,)), ((), ()))


SPI1 = 2
SPI2 = 2


def _step(emb_blk, h_prev, wih_ref, whh_ref, bias_ref):
    acc = lax.dot_general(emb_blk.astype(jnp.bfloat16), wih_ref[...],
                          _NT, preferred_element_type=jnp.float32)
    acc = acc + lax.dot_general(h_prev, whh_ref[...], _NT,
                                preferred_element_type=jnp.float32)
    return jnp.tanh(acc + bias_ref[...])


def _rnn_seg1_body(emb_ref, wih_ref, whh_ref, bias_ref, out_ref, h_ref):
    t = pl.program_id(0)
    h = jnp.where(t > 0, h_ref[...], jnp.zeros((B, HIDDEN), jnp.bfloat16))
    for s in range(SPI1):
        h = _step(emb_ref[pl.ds(s * B, B), :], h, wih_ref, whh_ref,
                  bias_ref).astype(jnp.bfloat16)
    h_ref[...] = h

    @pl.when(t == SPLIT // SPI1 - 1)
    def _():
        out_ref[...] = h


def _rnn_seg2_body(emb_ref, h0_ref, wih_ref, whh_ref, bias_ref, fc1w_ref,
                   fc1b_ref, fc2w_ref, fc2b_ref, out_ref, h_ref):
    t = pl.program_id(0)
    T = (L - SPLIT) // SPI2
    h = jnp.where(t > 0, h_ref[...], h0_ref[...])
    for s in range(SPI2):
        h_new = _step(emb_ref[pl.ds(s * B, B), :], h, wih_ref, whh_ref,
                      bias_ref)
        h = h_new.astype(jnp.bfloat16)
    h_ref[...] = h

    @pl.when(t == T - 1)
    def _():
        feat = lax.dot_general(h_new, fc1w_ref[...], _NT,
                               preferred_element_type=jnp.float32)
        feat = jnp.maximum(feat + fc1b_ref[...], 0.0)
        logits = lax.dot_general(feat, fc2w_ref[...], _NT,
                                 preferred_element_type=jnp.float32)
        logits = logits + fc2b_ref[...]
        m = jnp.max(logits, axis=1, keepdims=True)
        e = jnp.exp(logits - m)
        p = e / jnp.sum(e, axis=1, keepdims=True)
        out_ref[...] = p[:, :OUT]


_FULL = lambda t: (0, 0)


def _rnn_seg1(emb1, wih, whh, bias):
    return pl.pallas_call(
        _rnn_seg1_body,
        grid=(SPLIT // SPI1,),
        in_specs=[
            pl.BlockSpec((SPI1 * B, EMBED), lambda t: (t, 0)),
            pl.BlockSpec((HIDDEN, EMBED), _FULL),
            pl.BlockSpec((HIDDEN, HIDDEN), _FULL),
            pl.BlockSpec((1, HIDDEN), _FULL),
        ],
        out_specs=pl.BlockSpec((B, HIDDEN), _FULL),
        out_shape=jax.ShapeDtypeStruct((B, HIDDEN), jnp.bfloat16),
        scratch_shapes=[pltpu.VMEM((B, HIDDEN), jnp.bfloat16)],
        compiler_params=pltpu.CompilerParams(
            dimension_semantics=("arbitrary",)),
    )(emb1, wih, whh, bias)


def _rnn_seg2(emb2, h0, wih, whh, bias, fc1w_t, fc1b, fc2w_pad, fc2b_pad):
    return pl.pallas_call(
        _rnn_seg2_body,
        grid=((L - SPLIT) // SPI2,),
        in_specs=[
            pl.BlockSpec((SPI2 * B, EMBED), lambda t: (t, 0)),
            pl.BlockSpec((B, HIDDEN), _FULL),
            pl.BlockSpec((HIDDEN, EMBED), _FULL),
            pl.BlockSpec((HIDDEN, HIDDEN), _FULL),
            pl.BlockSpec((1, HIDDEN), _FULL),
            pl.BlockSpec((FC1, HIDDEN), _FULL),
            pl.BlockSpec((1, FC1), _FULL),
            pl.BlockSpec((LANE, FC1), _FULL),
            pl.BlockSpec((1, LANE), _FULL),
        ],
        out_specs=pl.BlockSpec((B, OUT), _FULL),
        out_shape=jax.ShapeDtypeStruct((B, OUT), jnp.float32),
        scratch_shapes=[pltpu.VMEM((B, HIDDEN), jnp.bfloat16)],
        compiler_params=pltpu.CompilerParams(
            dimension_semantics=("arbitrary",)),
    )(emb2, h0, wih, whh, bias, fc1w_t, fc1b, fc2w_pad, fc2b_pad)


def kernel(x, embed_table, W_ih, b_ih, W_hh, b_hh, fc1_W, fc1_b, fc2_W, fc2_b):
    idx = jnp.swapaxes(x, 0, 1).reshape(-1).astype(jnp.int32)
    idx1 = idx[: SPLIT * B]
    idx2 = idx[SPLIT * B:]
    emb1 = _make_sc_gather(VOCAB, EMBED, SPLIT * B)(embed_table, idx1)
    emb2 = _make_sc_gather(VOCAB, EMBED, (L - SPLIT) * B)(embed_table, idx2)

    wih = W_ih.astype(jnp.bfloat16)
    whh = W_hh.astype(jnp.bfloat16)
    bias = (b_ih + b_hh).reshape(1, HIDDEN)
    fc2w_pad = jnp.pad(fc2_W, ((0, LANE - OUT), (0, 0)))
    fc2b_pad = jnp.pad(fc2_b, (0, LANE - OUT),
                       constant_values=-1e30).reshape(1, LANE)

    h_mid = _rnn_seg1(emb1, wih, whh, bias)
    return _rnn_seg2(emb2, h_mid, wih, whh, bias, fc1_W,
                     fc1_b.reshape(1, FC1), fc2w_pad, fc2b_pad)
